# baseline (speedup 1.0000x reference)
import functools

import jax
import jax.numpy as jnp
from jax import lax
from jax.experimental import pallas as pl
from jax.experimental.pallas import tpu as pltpu
from jax.experimental.pallas import tpu_sc as plsc

N = 50000
D = 128
NC = 2
NS = 16
BE = 128
SB = 2048
CH = 8448
NCHUNK = 6
NPAD = CH * NCHUNK
PASSES = NCHUNK // NC
CPT = CH // NS
ZR = 66
BR = 1000


def _pad_edges(ei, w):
    e = ei.shape[1]
    ept = ((e + NS * SB - 1) // (NS * SB)) * SB
    pad = ept * NS - e
    src = jnp.concatenate([ei[0], jnp.zeros((pad,), jnp.int32)])
    dst = jnp.concatenate([ei[1], jnp.zeros((pad,), jnp.int32)])
    wp = jnp.concatenate([w, jnp.zeros((pad,), jnp.float32)])
    return src, dst, wp, ept


@functools.cache
def _make_segsum(ept):
    nsb = ept // SB
    nblk = SB // BE
    mesh = plsc.VectorSubcoreMesh(core_axis_name="c", subcore_axis_name="s")

    @functools.partial(
        pl.kernel,
        mesh=mesh,
        compiler_params=pltpu.CompilerParams(needs_layout_passes=False),
        out_type=jax.ShapeDtypeStruct((NPAD, D), jnp.float32),
        scratch_types=[
            pltpu.VMEM_SHARED((CH, D), jnp.float32),
            pltpu.VMEM((ZR, D), jnp.float32),
            pltpu.VMEM((2, SB), jnp.int32),
            pltpu.VMEM((2, SB), jnp.int32),
            pltpu.VMEM((2, SB), jnp.float32),
            pltpu.VMEM((2, BE), jnp.int32),
            pltpu.VMEM((2, BE), jnp.int32),
            pltpu.VMEM((2, BE), jnp.float32),
            pltpu.VMEM((BE,), jnp.int32),
            pltpu.VMEM((BE,), jnp.int32),
            pltpu.VMEM((BE,), jnp.float32),
            pltpu.VMEM((2, BE, D), jnp.float32),
            pltpu.SemaphoreType.DMA((2,)),
            pltpu.SemaphoreType.DMA,
        ],
    )
    def seg(h_hbm, src_hbm, dst_hbm, w_hbm, out_hbm,
            acc, zbuf, ebs, ebd, ebw,
            sts, std, stw, ovs, ovd, ovw, rows, esem, gsem):
        c = lax.axis_index("c")
        s = lax.axis_index("s")
        zero16f = jnp.zeros((16,), jnp.float32)
        zero16i = jnp.zeros((16,), jnp.int32)
        iota16 = lax.iota(jnp.int32, 16)

        def zb(i, carry):
            for u in range(8):
                zbuf[i, pl.ds(u * 16, 16)] = zero16f
            return carry

        lax.fori_loop(0, ZR, zb, 0)
        for k in range(2):
            for g in range(8):
                sl = pl.ds(g * 16, 16)
                sts[k, sl] = zero16i
                std[k, sl] = zero16i
                stw[k, sl] = zero16f
        for g in range(8):
            sl = pl.ds(g * 16, 16)
            ovs[sl] = zero16i
            ovd[sl] = zero16i
            ovw[sl] = zero16f

        e0t = s * ept

        def load_sb(t, k):
            e0 = e0t + t * SB
            sem = esem.at[k]
            pltpu.async_copy(src_hbm.at[pl.ds(e0, SB)], ebs.at[k], sem)
            pltpu.async_copy(dst_hbm.at[pl.ds(e0, SB)], ebd.at[k], sem)
            pltpu.async_copy(w_hbm.at[pl.ds(e0, SB)], ebw.at[k], sem)

        def wait_sb(t, k):
            e0 = e0t + t * SB
            sem = esem.at[k]
            pltpu.make_async_copy(
                src_hbm.at[pl.ds(e0, SB)], ebs.at[k], sem).wait()
            pltpu.make_async_copy(
                dst_hbm.at[pl.ds(e0, SB)], ebd.at[k], sem).wait()
            pltpu.make_async_copy(
                w_hbm.at[pl.ds(e0, SB)], ebw.at[k], sem).wait()

        def start_gather(k):
            pltpu.async_copy(h_hbm.at[sts.at[k]], rows.at[k], gsem)

        def process(k):
            pltpu.make_async_copy(
                h_hbm.at[sts.at[k]], rows.at[k], gsem).wait()

            def scale(j, carry2):
                wspl = plsc.load_gather(
                    stw, [jnp.full((16,), k, jnp.int32),
                          jnp.full((16,), j, jnp.int32)])
                for u in range(8):
                    sl2 = pl.ds(u * 16, 16)
                    rows[k, j, sl2] = rows[k, j, sl2] * wspl
                return carry2

            lax.fori_loop(0, BE, scale, 0)
            pltpu.sync_copy(rows.at[k], acc.at[std.at[k]], add=True)

        for p in range(PASSES):
            q = c * PASSES + p
            base = q * CH
            for k in range(CPT // ZR):
                pltpu.sync_copy(zbuf, acc.at[pl.ds(s * CPT + k * ZR, ZR)])
            plsc.subcore_barrier()
            load_sb(0, 0)

            def sblock(t, st):
                kcur = t & 1

                @pl.when(t + 1 < nsb)
                def _():
                    load_sb(t + 1, 1 - kcur)

                wait_sb(t, kcur)

                def eblk(b, st2):
                    cnt, par, pend = st2
                    boff = b * BE
                    for g in range(8):
                        sl = pl.ds(boff + g * 16, 16)
                        dv = ebd[kcur, sl]
                        sv = ebs[kcur, sl]
                        wv = ebw[kcur, sl]
                        inm = (dv >= base) & (dv < base + CH)
                        ones = jnp.where(inm, 1, 0).astype(jnp.int32)
                        pos = cnt + plsc.cumsum(ones) - 1
                        posm = pos & (BE - 1)
                        in_a = inm & (pos < BE)
                        in_b = inm & (pos >= BE)
                        dloc = dv - base
                        parv = jnp.full((16,), par, jnp.int32)
                        plsc.store_scatter(sts, [parv, posm], sv, mask=in_a)
                        plsc.store_scatter(std, [parv, posm], dloc, mask=in_a)
                        plsc.store_scatter(stw, [parv, posm], wv, mask=in_a)
                        plsc.store_scatter(ovs, [posm], sv, mask=in_b)
                        plsc.store_scatter(ovd, [posm], dloc, mask=in_b)
                        plsc.store_scatter(ovw, [posm], wv, mask=in_b)
                        cnt = cnt + plsc.all_reduce_population_count(inm)

                    def fire(args):
                        cnt2, par2, pend2 = args
                        opar = 1 - par2

                        @pl.when(pend2 == 1)
                        def _():
                            process(opar)

                        start_gather(par2)
                        for g2 in range(8):
                            sl2 = pl.ds(g2 * 16, 16)
                            sts[opar, sl2] = ovs[sl2]
                            std[opar, sl2] = ovd[sl2]
                            stw[opar, sl2] = ovw[sl2]
                        return (cnt2 - BE, opar, jnp.int32(1))

                    st2 = lax.cond(jnp.max(cnt) >= BE, fire,
                                   lambda a: a, (cnt, par, pend))
                    return st2

                return lax.fori_loop(0, nblk, eblk, st)

            cnt, par, pend = lax.fori_loop(
                0, nsb, sblock,
                (zero16i, jnp.int32(0), jnp.int32(0)))

            @pl.when(pend == 1)
            def _():
                process(1 - par)

            for g in range(8):
                sl = pl.ds(g * 16, 16)
                lane = iota16 + g * 16
                stw[par, sl] = jnp.where(lane < cnt, stw[par, sl], 0.0)
            start_gather(par)
            process(par)
            plsc.subcore_barrier()
            pltpu.sync_copy(
                acc.at[pl.ds(s * CPT, CPT)],
                out_hbm.at[pl.ds(base + s * CPT, CPT)])
            plsc.subcore_barrier()

    return seg


def _mm(xs, ws, b, act):
    nin = len(xs)

    def body(*refs):
        in_refs = refs[:nin]
        w_refs = refs[nin:2 * nin]
        b_ref = refs[2 * nin]
        o_ref = refs[2 * nin + 1]
        acc = jnp.zeros((BR, D), jnp.float32)
        for xr, wr in zip(in_refs, w_refs):
            acc = acc + jnp.dot(xr[...], wr[...],
                                preferred_element_type=jnp.float32)
        acc = acc + b_ref[...]
        if act == "relu":
            acc = jnp.maximum(acc, 0.0)
        elif act == "lrelu":
            acc = jnp.where(acc > 0, acc, acc * 0.01)
        o_ref[...] = acc

    in_specs = (
        [pl.BlockSpec((BR, D), lambda i: (i, 0)) for _ in xs]
        + [pl.BlockSpec((D, D), lambda i: (0, 0)) for _ in ws]
        + [pl.BlockSpec((1, D), lambda i: (0, 0))]
    )
    f = pl.pallas_call(
        body,
        grid=(N // BR,),
        in_specs=in_specs,
        out_specs=pl.BlockSpec((BR, D), lambda i: (i, 0)),
        out_shape=jax.ShapeDtypeStruct((N, D), jnp.float32),
    )
    return f(*xs, *ws, b.reshape(1, D))


def kernel(x_protocol, x_impression, x_treatment, edge_index_has,
           edge_index_suggests, edge_index_indicates, edge_index_issuggestedby,
           edge_weight_has, edge_weight_suggests, edge_weight_indicates,
           edge_weight_issuggestedby, params):
    lin = params["lin"]
    h = {
        "protocol": _mm([x_protocol], [lin["protocol"]["W"]],
                        lin["protocol"]["b"], "relu"),
        "impression": _mm([x_impression], [lin["impression"]["W"]],
                          lin["impression"]["b"], "relu"),
        "treatment": _mm([x_treatment], [lin["treatment"]["W"]],
                         lin["treatment"]["b"], "relu"),
    }
    edges = {
        "has": _pad_edges(edge_index_has, edge_weight_has),
        "suggests": _pad_edges(edge_index_suggests, edge_weight_suggests),
        "indicates": _pad_edges(edge_index_indicates, edge_weight_indicates),
        "issuggestedby": _pad_edges(edge_index_issuggestedby,
                                    edge_weight_issuggestedby),
    }
    seg = _make_segsum(edges["has"][3])
    src_of = {"has": "protocol", "suggests": "protocol",
              "indicates": "impression", "issuggestedby": "treatment"}
    for layer in params["convs"]:
        agg = {et: seg(h[src_of[et]], *edges[et][:3]) for et in edges}
        new_i = _mm([agg["has"], h["impression"]],
                    [layer["has"]["W_rel"], layer["has"]["W_root"]],
                    layer["has"]["b_rel"], "lrelu")
        new_t = _mm([agg["suggests"], h["treatment"]],
                    [layer["suggests"]["W_rel"], layer["suggests"]["W_root"]],
                    layer["suggests"]["b_rel"], "lrelu")
        new_p = _mm(
            [agg["indicates"], agg["issuggestedby"], h["protocol"]],
            [layer["indicates"]["W_rel"], layer["issuggestedby"]["W_rel"],
             layer["indicates"]["W_root"] + layer["issuggestedby"]["W_root"]],
            layer["indicates"]["b_rel"] + layer["issuggestedby"]["b_rel"],
            "lrelu")
        h = {"protocol": new_p, "impression": new_i, "treatment": new_t}
    return _mm([h["protocol"]], [params["out"]["W"]], params["out"]["b"], None)


# speedup vs baseline: 1.2935x; 1.2935x over previous
import functools

import jax
import jax.numpy as jnp
from jax import lax
from jax.experimental import pallas as pl
from jax.experimental.pallas import tpu as pltpu
from jax.experimental.pallas import tpu_sc as plsc

N = 50000
D = 128
NC = 2
NS = 16
BE = 128
SB = 2048
CH = 8448
NCHUNK = 6
NPAD = CH * NCHUNK
PASSES = NCHUNK // NC
CPT = CH // NS
ZR = 66
BR = 1000


def _pad_edges(ei, w):
    e = ei.shape[1]
    ept = ((e + NS * 2 * SB - 1) // (NS * 2 * SB)) * 2 * SB
    pad = ept * NS - e
    src = jnp.concatenate([ei[0], jnp.zeros((pad,), jnp.int32)])
    dst = jnp.concatenate([ei[1], jnp.zeros((pad,), jnp.int32)])
    wp = jnp.concatenate([w, jnp.zeros((pad,), jnp.float32)])
    return src, dst, wp, ept


@functools.cache
def _make_segsum(ept):
    nsb = ept // SB
    npair = nsb // 2
    nblk = SB // BE
    mesh = plsc.VectorSubcoreMesh(core_axis_name="c", subcore_axis_name="s")

    @functools.partial(
        pl.kernel,
        mesh=mesh,
        compiler_params=pltpu.CompilerParams(needs_layout_passes=False),
        out_type=jax.ShapeDtypeStruct((NPAD, D), jnp.float32),
        scratch_types=[
            pltpu.VMEM_SHARED((CH, D), jnp.float32),
            pltpu.VMEM((ZR, D), jnp.float32),
            pltpu.VMEM((SB,), jnp.int32),
            pltpu.VMEM((SB,), jnp.int32),
            pltpu.VMEM((SB,), jnp.float32),
            pltpu.VMEM((SB,), jnp.int32),
            pltpu.VMEM((SB,), jnp.int32),
            pltpu.VMEM((SB,), jnp.float32),
            pltpu.VMEM((BE,), jnp.int32),
            pltpu.VMEM((BE,), jnp.int32),
            pltpu.VMEM((BE,), jnp.float32),
            pltpu.VMEM((BE,), jnp.int32),
            pltpu.VMEM((BE,), jnp.int32),
            pltpu.VMEM((BE,), jnp.float32),
            pltpu.VMEM((BE,), jnp.int32),
            pltpu.VMEM((BE,), jnp.int32),
            pltpu.VMEM((BE,), jnp.float32),
            pltpu.VMEM((BE,), jnp.int32),
            pltpu.VMEM((BE,), jnp.int32),
            pltpu.VMEM((BE,), jnp.float32),
            pltpu.VMEM((BE, D), jnp.float32),
            pltpu.VMEM((BE, D), jnp.float32),
            pltpu.SemaphoreType.DMA,
            pltpu.SemaphoreType.DMA,
            pltpu.SemaphoreType.DMA,
            pltpu.SemaphoreType.DMA,
        ],
    )
    def seg(h_hbm, src_hbm, dst_hbm, w_hbm, out_hbm,
            acc, zbuf, ebs0, ebd0, ebw0, ebs1, ebd1, ebw1,
            sts, std, stw, ovs, ovd, ovw,
            gs0, gd0, gw0, gs1, gd1, gw1, rows0, rows1,
            esem0, esem1, gsem0, gsem1):
        c = lax.axis_index("c")
        s = lax.axis_index("s")
        zero16f = jnp.zeros((16,), jnp.float32)
        zero16i = jnp.zeros((16,), jnp.int32)
        iota16 = lax.iota(jnp.int32, 16)

        def zb(i, carry):
            for u in range(8):
                zbuf[i, pl.ds(u * 16, 16)] = zero16f
            return carry

        lax.fori_loop(0, ZR, zb, 0)
        for g in range(8):
            sl = pl.ds(g * 16, 16)
            sts[sl] = zero16i
            std[sl] = zero16i
            stw[sl] = zero16f
            ovs[sl] = zero16i
            ovd[sl] = zero16i
            ovw[sl] = zero16f

        e0t = s * ept
        ebufs = ((ebs0, ebd0, ebw0, esem0), (ebs1, ebd1, ebw1, esem1))
        gsets = ((gs0, gd0, gw0, rows0, gsem0), (gs1, gd1, gw1, rows1, gsem1))

        def load_sb(t, half):
            bs, bd, bw, sem = ebufs[half]
            e0 = e0t + t * SB
            pltpu.async_copy(src_hbm.at[pl.ds(e0, SB)], bs, sem)
            pltpu.async_copy(dst_hbm.at[pl.ds(e0, SB)], bd, sem)
            pltpu.async_copy(w_hbm.at[pl.ds(e0, SB)], bw, sem)

        def wait_sb(t, half):
            bs, bd, bw, sem = ebufs[half]
            e0 = e0t + t * SB
            pltpu.make_async_copy(src_hbm.at[pl.ds(e0, SB)], bs, sem).wait()
            pltpu.make_async_copy(dst_hbm.at[pl.ds(e0, SB)], bd, sem).wait()
            pltpu.make_async_copy(w_hbm.at[pl.ds(e0, SB)], bw, sem).wait()

        def start_gather(k):
            gs, gd, gw, rows, sem = gsets[k]
            pltpu.async_copy(h_hbm.at[gs], rows, sem)

        def process(k):
            gs, gd, gw, rows, sem = gsets[k]
            pltpu.make_async_copy(h_hbm.at[gs], rows, sem).wait()

            def scale(j, carry2):
                wspl = plsc.load_gather(gw, [jnp.full((16,), j, jnp.int32)])
                for u in range(8):
                    sl2 = pl.ds(u * 16, 16)
                    rows[j, sl2] = rows[j, sl2] * wspl
                return carry2

            lax.fori_loop(0, BE, scale, 0)
            pltpu.sync_copy(rows, acc.at[gd], add=True)

        def stage_to_set(k):
            gs, gd, gw, rows, sem = gsets[k]
            for g in range(8):
                sl = pl.ds(g * 16, 16)
                gs[sl] = sts[sl]
                gd[sl] = std[sl]
                gw[sl] = stw[sl]

        def ovf_to_stage():
            for g in range(8):
                sl = pl.ds(g * 16, 16)
                sts[sl] = ovs[sl]
                std[sl] = ovd[sl]
                stw[sl] = ovw[sl]

        def make_fire(k):
            def fire_k(st):
                cnt, fp, pd0, pd1 = st
                pend = pd0 if k == 0 else pd1

                @pl.when(pend == 1)
                def _():
                    process(k)

                stage_to_set(k)
                start_gather(k)
                ovf_to_stage()
                if k == 0:
                    return (cnt - BE, jnp.int32(1), jnp.int32(1), pd1)
                return (cnt - BE, jnp.int32(0), pd0, jnp.int32(1))
            return fire_k

        fire0 = make_fire(0)
        fire1 = make_fire(1)

        for p in range(PASSES):
            q = c * PASSES + p
            base = q * CH
            for k in range(CPT // ZR):
                pltpu.sync_copy(zbuf, acc.at[pl.ds(s * CPT + k * ZR, ZR)])
            plsc.subcore_barrier()

            def scan_buf(half, st):
                bs, bd, bw, _ = ebufs[half]

                def eblk(b, st2):
                    cnt, fp, pd0, pd1 = st2
                    boff = b * BE
                    for g in range(8):
                        sl = pl.ds(boff + g * 16, 16)
                        dv = bd[sl]
                        sv = bs[sl]
                        wv = bw[sl]
                        inm = (dv >= base) & (dv < base + CH)
                        ones = jnp.where(inm, 1, 0).astype(jnp.int32)
                        pos = cnt + plsc.cumsum(ones) - 1
                        posm = pos & (BE - 1)
                        in_a = inm & (pos < BE)
                        in_b = inm & (pos >= BE)
                        dloc = dv - base
                        plsc.store_scatter(sts, [posm], sv, mask=in_a)
                        plsc.store_scatter(std, [posm], dloc, mask=in_a)
                        plsc.store_scatter(stw, [posm], wv, mask=in_a)
                        plsc.store_scatter(ovs, [posm], sv, mask=in_b)
                        plsc.store_scatter(ovd, [posm], dloc, mask=in_b)
                        plsc.store_scatter(ovw, [posm], wv, mask=in_b)
                        cnt = cnt + plsc.all_reduce_population_count(inm)

                    st2 = (cnt, fp, pd0, pd1)

                    def fire(stf):
                        return lax.cond(stf[1] == 0, fire0, fire1, stf)

                    return lax.cond(jnp.max(cnt) >= BE, fire,
                                    lambda a: a, st2)

                return lax.fori_loop(0, nblk, eblk, st)

            load_sb(0, 0)

            def sbpair(t2, st):
                t0 = 2 * t2
                load_sb(t0 + 1, 1)
                wait_sb(t0, 0)
                st = scan_buf(0, st)

                @pl.when(t2 + 1 < npair)
                def _():
                    load_sb(t0 + 2, 0)

                wait_sb(t0 + 1, 1)
                return scan_buf(1, st)

            cnt, fp, pd0, pd1 = lax.fori_loop(
                0, npair, sbpair,
                (zero16i, jnp.int32(0), jnp.int32(0), jnp.int32(0)))

            @pl.when(pd0 == 1)
            def _():
                process(0)

            @pl.when(pd1 == 1)
            def _():
                process(1)

            for g in range(8):
                sl = pl.ds(g * 16, 16)
                lane = iota16 + g * 16
                stw[sl] = jnp.where(lane < cnt, stw[sl], 0.0)
            stage_to_set(0)
            start_gather(0)
            process(0)
            plsc.subcore_barrier()
            pltpu.sync_copy(
                acc.at[pl.ds(s * CPT, CPT)],
                out_hbm.at[pl.ds(base + s * CPT, CPT)])
            plsc.subcore_barrier()

    return seg


def _mm(xs, ws, b, act):
    nin = len(xs)

    def body(*refs):
        in_refs = refs[:nin]
        w_refs = refs[nin:2 * nin]
        b_ref = refs[2 * nin]
        o_ref = refs[2 * nin + 1]
        acc = jnp.zeros((BR, D), jnp.float32)
        for xr, wr in zip(in_refs, w_refs):
            acc = acc + jnp.dot(xr[...], wr[...],
                                preferred_element_type=jnp.float32)
        acc = acc + b_ref[...]
        if act == "relu":
            acc = jnp.maximum(acc, 0.0)
        elif act == "lrelu":
            acc = jnp.where(acc > 0, acc, acc * 0.01)
        o_ref[...] = acc

    in_specs = (
        [pl.BlockSpec((BR, D), lambda i: (i, 0)) for _ in xs]
        + [pl.BlockSpec((D, D), lambda i: (0, 0)) for _ in ws]
        + [pl.BlockSpec((1, D), lambda i: (0, 0))]
    )
    f = pl.pallas_call(
        body,
        grid=(N // BR,),
        in_specs=in_specs,
        out_specs=pl.BlockSpec((BR, D), lambda i: (i, 0)),
        out_shape=jax.ShapeDtypeStruct((N, D), jnp.float32),
    )
    return f(*xs, *ws, b.reshape(1, D))


def kernel(x_protocol, x_impression, x_treatment, edge_index_has,
           edge_index_suggests, edge_index_indicates, edge_index_issuggestedby,
           edge_weight_has, edge_weight_suggests, edge_weight_indicates,
           edge_weight_issuggestedby, params):
    lin = params["lin"]
    h = {
        "protocol": _mm([x_protocol], [lin["protocol"]["W"]],
                        lin["protocol"]["b"], "relu"),
        "impression": _mm([x_impression], [lin["impression"]["W"]],
                          lin["impression"]["b"], "relu"),
        "treatment": _mm([x_treatment], [lin["treatment"]["W"]],
                         lin["treatment"]["b"], "relu"),
    }
    edges = {
        "has": _pad_edges(edge_index_has, edge_weight_has),
        "suggests": _pad_edges(edge_index_suggests, edge_weight_suggests),
        "indicates": _pad_edges(edge_index_indicates, edge_weight_indicates),
        "issuggestedby": _pad_edges(edge_index_issuggestedby,
                                    edge_weight_issuggestedby),
    }
    seg = _make_segsum(edges["has"][3])
    src_of = {"has": "protocol", "suggests": "protocol",
              "indicates": "impression", "issuggestedby": "treatment"}
    for layer in params["convs"]:
        agg = {et: seg(h[src_of[et]], *edges[et][:3]) for et in edges}
        new_i = _mm([agg["has"], h["impression"]],
                    [layer["has"]["W_rel"], layer["has"]["W_root"]],
                    layer["has"]["b_rel"], "lrelu")
        new_t = _mm([agg["suggests"], h["treatment"]],
                    [layer["suggests"]["W_rel"], layer["suggests"]["W_root"]],
                    layer["suggests"]["b_rel"], "lrelu")
        new_p = _mm(
            [agg["indicates"], agg["issuggestedby"], h["protocol"]],
            [layer["indicates"]["W_rel"], layer["issuggestedby"]["W_rel"],
             layer["indicates"]["W_root"] + layer["issuggestedby"]["W_root"]],
            layer["indicates"]["b_rel"] + layer["issuggestedby"]["b_rel"],
            "lrelu")
        h = {"protocol": new_p, "impression": new_i, "treatment": new_t}
    return _mm([h["protocol"]], [params["out"]["W"]], params["out"]["b"], None)
